# Initial kernel scaffold; baseline (speedup 1.0000x reference)
import jax
import jax.numpy as jnp
from jax.experimental import pallas as pl


def kernel(hidden_states, Wr, Wg, Wu, Wd):
    raise NotImplementedError("write your pallas kernel here")


# speedup vs baseline: 1.0367x; 1.0367x over previous
import functools

import jax
import jax.numpy as jnp
from jax.experimental import pallas as pl

B, S, H, I, E, TOP_K = 1, 2048, 768, 2048, 8, 2
T = B * S
TM = 256


def _router_body(x_ref, wr_ref, comb_ref):
    x = x_ref[...]
    logits = jnp.dot(x, wr_ref[...], preferred_element_type=jnp.float32)
    lmax = jnp.max(logits, axis=1, keepdims=True)
    z = jnp.exp(logits - lmax)
    iota = jax.lax.broadcasted_iota(jnp.int32, (TM, E), 1)
    e1 = jnp.argmax(z, axis=1)[:, None]
    m1 = jnp.max(z, axis=1, keepdims=True)
    z2 = jnp.where(iota == e1, -jnp.inf, z)
    e2 = jnp.argmax(z2, axis=1)[:, None]
    m2 = jnp.max(z2, axis=1, keepdims=True)
    sel = (iota == e1) | (iota == e2)
    comb_ref[...] = jnp.where(sel, z / (m1 + m2), 0.0)


def _moe_body(xb_ref, comb_ref, wg_ref, wu_ref, wd_ref, out_ref):
    e = pl.program_id(1)
    xb = xb_ref[...]
    gate = jnp.dot(xb, wg_ref[...], preferred_element_type=jnp.float32)
    up = jnp.dot(xb, wu_ref[...], preferred_element_type=jnp.float32)
    inter = up * gate * jax.nn.sigmoid(gate)
    y = jnp.dot(inter.astype(jnp.bfloat16), wd_ref[...],
                preferred_element_type=jnp.float32)
    iota = jax.lax.broadcasted_iota(jnp.int32, (TM, E), 1)
    c = jnp.sum(jnp.where(iota == e, comb_ref[...], 0.0), axis=1, keepdims=True)
    contrib = c * y

    @pl.when(e == 0)
    def _():
        out_ref[...] = contrib

    @pl.when(e != 0)
    def _():
        out_ref[...] += contrib


def kernel(hidden_states, Wr, Wg, Wu, Wd):
    b, s, h = hidden_states.shape
    x = hidden_states.reshape(T, H)

    combine = pl.pallas_call(
        _router_body,
        grid=(T // TM,),
        in_specs=[
            pl.BlockSpec((TM, H), lambda i: (i, 0)),
            pl.BlockSpec((H, E), lambda i: (0, 0)),
        ],
        out_specs=pl.BlockSpec((TM, E), lambda i: (i, 0)),
        out_shape=jax.ShapeDtypeStruct((T, E), jnp.float32),
    )(x, Wr)

    xb = x.astype(jnp.bfloat16)
    Wg_b = Wg.astype(jnp.bfloat16)
    Wu_b = Wu.astype(jnp.bfloat16)
    Wd_b = Wd.astype(jnp.bfloat16)

    out = pl.pallas_call(
        _moe_body,
        grid=(T // TM, E),
        in_specs=[
            pl.BlockSpec((TM, H), lambda i, e: (i, 0)),
            pl.BlockSpec((TM, E), lambda i, e: (i, 0)),
            pl.BlockSpec((None, H, I), lambda i, e: (e, 0, 0)),
            pl.BlockSpec((None, H, I), lambda i, e: (e, 0, 0)),
            pl.BlockSpec((None, I, H), lambda i, e: (e, 0, 0)),
        ],
        out_specs=pl.BlockSpec((TM, H), lambda i, e: (i, 0)),
        out_shape=jax.ShapeDtypeStruct((T, H), jnp.float32),
    )(xb, combine, Wg_b, Wu_b, Wd_b)

    return out.reshape(b, s, h)
